# baseline (speedup 1.0000x reference)
import jax
from jax.experimental import pallas as pl
from jax.experimental.pallas import tpu as pltpu

_ROWS = 4 * 1024
_COLS = 768
_N = 16
_WIN = 4
_CR = _ROWS // _N


def _copy(x_ref, o_ref, buf, in_sems, out_sems):
    def in_cp(i):
        sl = pl.ds(i * _CR, _CR)
        return pltpu.make_async_copy(x_ref.at[sl], buf.at[sl], in_sems.at[i])

    def out_cp(i):
        sl = pl.ds(i * _CR, _CR)
        return pltpu.make_async_copy(buf.at[sl], o_ref.at[sl], out_sems.at[i])

    for i in range(_WIN):
        in_cp(i).start()
    for i in range(_N):
        in_cp(i).wait()
        out_cp(i).start()
        if i + _WIN < _N:
            in_cp(i + _WIN).start()
    for i in range(_N):
        out_cp(i).wait()


def kernel(x, H, W):
    x2 = x.reshape(_ROWS, _COLS)
    y = pl.pallas_call(
        _copy,
        out_shape=jax.ShapeDtypeStruct((_ROWS, _COLS), x.dtype),
        in_specs=[pl.BlockSpec(memory_space=pl.ANY)],
        out_specs=pl.BlockSpec(memory_space=pl.ANY),
        scratch_shapes=[
            pltpu.VMEM((_ROWS, _COLS), x.dtype),
            pltpu.SemaphoreType.DMA((_N,)),
            pltpu.SemaphoreType.DMA((_N,)),
        ],
    )(x2)
    return (y.reshape(x.shape), H, W)


# speedup vs baseline: 1.1548x; 1.1548x over previous
import jax
from jax.experimental import pallas as pl
from jax.experimental.pallas import tpu as pltpu

_ROWS = 4 * 1024
_COLS = 768
_NCHUNK = 8
_CROWS = _ROWS // _NCHUNK


def _identity_copy(x_ref, o_ref, buf, in_sems, out_sems):
    for i in range(_NCHUNK):
        sl = pl.ds(i * _CROWS, _CROWS)
        pltpu.make_async_copy(x_ref.at[sl], buf.at[sl], in_sems.at[i]).start()
    for i in range(_NCHUNK):
        sl = pl.ds(i * _CROWS, _CROWS)
        pltpu.make_async_copy(x_ref.at[sl], buf.at[sl], in_sems.at[i]).wait()
        pltpu.make_async_copy(buf.at[sl], o_ref.at[sl], out_sems.at[i]).start()
    for i in range(_NCHUNK):
        sl = pl.ds(i * _CROWS, _CROWS)
        pltpu.make_async_copy(buf.at[sl], o_ref.at[sl], out_sems.at[i]).wait()


def kernel(x, H, W):
    x2 = x.reshape(_ROWS, _COLS)
    y = pl.pallas_call(
        _identity_copy,
        out_shape=jax.ShapeDtypeStruct((_ROWS, _COLS), x.dtype),
        in_specs=[pl.BlockSpec(memory_space=pl.ANY)],
        out_specs=pl.BlockSpec(memory_space=pl.ANY),
        scratch_shapes=[
            pltpu.VMEM((_ROWS, _COLS), x.dtype),
            pltpu.SemaphoreType.DMA((_NCHUNK,)),
            pltpu.SemaphoreType.DMA((_NCHUNK,)),
        ],
    )(x2)
    return (y.reshape(x.shape), H, W)
